# baseline (speedup 1.0000x reference)
import functools

import jax
import jax.numpy as jnp
from jax import lax
from jax.experimental import pallas as pl
from jax.experimental.pallas import tpu as pltpu
from jax.experimental.pallas import tpu_sc as plsc

_N = 10000
_D = 128
_E = 320000
_F = 4
_TBLW = 2 * _F
_PKW = _TBLW // 2

_NC = 2
_NS = 16
_NW = _NC * _NS
_WSTRIDE = 9984
_CH = 5248
_NCHUNK = 2

_HIMASK = -65536


def _mm_body(w_ref, x_ref, o_ref):
    yt = jax.lax.dot_general(
        w_ref[...], x_ref[...],
        dimension_numbers=(((1,), (1,)), ((), ())),
        preferred_element_type=jnp.float32)
    b = jax.lax.bitcast_convert_type(yt, jnp.uint32) + jnp.uint32(0x8000)
    lo = b[:_PKW, :] >> 16
    hi = b[_PKW:, :] & jnp.uint32(0xFFFF0000)
    o_ref[...] = jax.lax.bitcast_convert_type(lo | hi, jnp.int32)


def _tanh16(a, b):
    e = jnp.exp((a + b) * 2.0)
    w = 1.0 / (e + 1.0)
    return 1.0 - (w + w)


def _lo(w):
    return plsc.bitcast(w << 16, jnp.float32)


def _hi(w):
    return plsc.bitcast(w & _HIMASK, jnp.float32)


_mesh = plsc.VectorSubcoreMesh(core_axis_name="c", subcore_axis_name="s")


@functools.partial(
    pl.kernel,
    mesh=_mesh,
    out_type=jax.ShapeDtypeStruct((_F * _E,), jnp.float32),
    compiler_params=pltpu.CompilerParams(needs_layout_passes=False),
    scratch_types=[
        pltpu.VMEM((_N * _PKW,), jnp.int32),
        pltpu.VMEM((_CH,), jnp.int32),
        pltpu.VMEM((_CH,), jnp.int32),
        pltpu.VMEM((_CH,), jnp.int32),
        pltpu.VMEM((_CH,), jnp.int32),
        pltpu.VMEM((_F * _CH,), jnp.float32),
        pltpu.VMEM((_F * _CH,), jnp.float32),
        pltpu.SemaphoreType.DMA,
        pltpu.SemaphoreType.DMA,
        pltpu.SemaphoreType.DMA,
        pltpu.SemaphoreType.DMA,
        pltpu.SemaphoreType.DMA,
    ],
)
def _edge_maps(ypk_hbm, ei_hbm, out_hbm, tbl_v, r0, c0, r1, c1, o0, o1,
               sem_t, sem_i0, sem_i1, sem_o0, sem_o1):
    wid = lax.axis_index("s") * _NC + lax.axis_index("c")
    base = wid * _WSTRIDE
    rbufs = (r0, r1)
    cbufs = (c0, c1)
    obufs = (o0, o1)
    isems = (sem_i0, sem_i1)
    osems = (sem_o0, sem_o1)

    tbl_dma = pltpu.async_copy(ypk_hbm, tbl_v, sem_t)

    def _start_idx(c):
        b = c % 2
        off = base + c * _CH
        hr = pltpu.async_copy(ei_hbm.at[pl.ds(off, _CH)], rbufs[b], isems[b])
        hc = pltpu.async_copy(ei_hbm.at[pl.ds(_E + off, _CH)], cbufs[b],
                              isems[b])
        return hr, hc

    idx_dmas = [None, None]
    out_dmas = [None, None]
    idx_dmas[0] = _start_idx(0)
    tbl_dma.wait()

    for c in range(_NCHUNK):
        b = c % 2
        off = base + c * _CH
        if c + 1 < _NCHUNK:
            idx_dmas[1 - b] = _start_idx(c + 1)
        for h in idx_dmas[b]:
            h.wait()
        if out_dmas[b] is not None:
            for h in out_dmas[b]:
                h.wait()
        rows_v = rbufs[b]
        cols_v = cbufs[b]
        out_v = obufs[b]

        @plsc.parallel_loop(0, _CH, 16, unroll=16)
        def _step(i):
            be = (i // 128) * 256 + (i % 128)
            r = rows_v[pl.ds(i, 16)]
            s = cols_v[pl.ds(i, 16)]
            w0 = plsc.load_gather(tbl_v, [r])
            w1 = plsc.load_gather(tbl_v, [r + _N])
            w2 = plsc.load_gather(tbl_v, [s + 2 * _N])
            w3 = plsc.load_gather(tbl_v, [s + 3 * _N])
            pairs = ((_lo(w0), _lo(w2)), (_hi(w0), _hi(w2)),
                     (_lo(w1), _lo(w3)), (_hi(w1), _hi(w3)))
            for j, (a, bb) in enumerate(pairs):
                out_v[pl.ds((j // 2) * (2 * _CH) + be + (j % 2) * 128, 16)] = (
                    _tanh16(a, bb))

        out_dmas[b] = tuple(
            pltpu.async_copy(out_v.at[pl.ds(j1 * 2 * _CH, 2 * _CH)],
                             out_hbm.at[pl.ds(j1 * 2 * _E + 2 * off, 2 * _CH)],
                             osems[b])
            for j1 in range(2))

    for hs in out_dmas:
        if hs is not None:
            for h in hs:
                h.wait()


def kernel(x, edge_index, W):
    w1t = W[:, :_D].T
    w2t = W[:, _D:].T
    wc = jnp.concatenate([w1t, w2t], axis=1)
    wct = wc[:, jnp.array([0, 2, 4, 6, 1, 3, 5, 7])].T
    ypk2 = pl.pallas_call(
        _mm_body,
        out_shape=jax.ShapeDtypeStruct((_PKW, _N), jnp.int32),
    )(wct, x)
    out_t = _edge_maps(ypk2.reshape(-1), edge_index.reshape(-1))
    return (out_t.reshape(2, _E // 128, 2, 128)
            .transpose(1, 3, 0, 2).reshape(_E, 2, 2))


# speedup vs baseline: 1.1836x; 1.1836x over previous
import functools

import jax
import jax.numpy as jnp
from jax import lax
from jax.experimental import pallas as pl
from jax.experimental.pallas import tpu as pltpu
from jax.experimental.pallas import tpu_sc as plsc

_N = 10000
_D = 128
_E = 320000
_F = 4
_TBLW = 2 * _F
_PKW = _TBLW // 2

_NC = 2
_NS = 16
_NW = _NC * _NS
_WSTRIDE = 9984
_CH = 5248
_NCHUNK = 2

_HIMASK = -65536


def _mm_body(w_ref, x_ref, o_ref):
    yt = jax.lax.dot_general(
        w_ref[...], x_ref[...],
        dimension_numbers=(((1,), (1,)), ((), ())),
        preferred_element_type=jnp.float32)
    b = jax.lax.bitcast_convert_type(yt, jnp.uint32) + jnp.uint32(0x8000)
    lo = b[:_PKW, :] >> 16
    hi = b[_PKW:, :] & jnp.uint32(0xFFFF0000)
    o_ref[...] = jax.lax.bitcast_convert_type(lo | hi, jnp.int32)


def _tanh16(a, b):
    e = jnp.exp((a + b) * 2.0)
    w = 1.0 / (e + 1.0)
    return 1.0 - (w + w)


def _lo(w):
    return plsc.bitcast(w << 16, jnp.float32)


def _hi(w):
    return plsc.bitcast(w & _HIMASK, jnp.float32)


_mesh = plsc.VectorSubcoreMesh(core_axis_name="c", subcore_axis_name="s")


@functools.partial(
    pl.kernel,
    mesh=_mesh,
    out_type=jax.ShapeDtypeStruct((_F * _E,), jnp.float32),
    compiler_params=pltpu.CompilerParams(needs_layout_passes=False),
    scratch_types=[
        pltpu.VMEM((_N * _PKW,), jnp.int32),
        pltpu.VMEM((_CH,), jnp.int32),
        pltpu.VMEM((_CH,), jnp.int32),
        pltpu.VMEM((_CH,), jnp.int32),
        pltpu.VMEM((_CH,), jnp.int32),
        pltpu.VMEM((_F * _CH,), jnp.float32),
        pltpu.VMEM((_F * _CH,), jnp.float32),
        pltpu.SemaphoreType.DMA,
        pltpu.SemaphoreType.DMA,
        pltpu.SemaphoreType.DMA,
        pltpu.SemaphoreType.DMA,
        pltpu.SemaphoreType.DMA,
    ],
)
def _edge_maps(ypk_hbm, ei_hbm, out_hbm, tbl_v, r0, c0, r1, c1, o0, o1,
               sem_t, sem_i0, sem_i1, sem_o0, sem_o1):
    wid = lax.axis_index("s") * _NC + lax.axis_index("c")
    base = wid * _WSTRIDE
    rbufs = (r0, r1)
    cbufs = (c0, c1)
    obufs = (o0, o1)
    isems = (sem_i0, sem_i1)
    osems = (sem_o0, sem_o1)

    tbl_dma = pltpu.async_copy(ypk_hbm, tbl_v, sem_t)

    def _start_idx(c):
        b = c % 2
        off = base + c * _CH
        hr = pltpu.async_copy(ei_hbm.at[pl.ds(off, _CH)], rbufs[b], isems[b])
        hc = pltpu.async_copy(ei_hbm.at[pl.ds(_E + off, _CH)], cbufs[b],
                              isems[b])
        return hr, hc

    idx_dmas = [None, None]
    out_dmas = [None, None]
    idx_dmas[0] = _start_idx(0)
    tbl_dma.wait()

    for c in range(_NCHUNK):
        b = c % 2
        off = base + c * _CH
        if c + 1 < _NCHUNK:
            idx_dmas[1 - b] = _start_idx(c + 1)
        for h in idx_dmas[b]:
            h.wait()
        if out_dmas[b] is not None:
            for h in out_dmas[b]:
                h.wait()
        rows_v = rbufs[b]
        cols_v = cbufs[b]
        out_v = obufs[b]

        @plsc.parallel_loop(0, _CH, 16, unroll=4)
        def _step(i):
            be = (i // 128) * 256 + (i % 128)
            r = rows_v[pl.ds(i, 16)]
            s = cols_v[pl.ds(i, 16)]
            w0 = plsc.load_gather(tbl_v, [r])
            w1 = plsc.load_gather(tbl_v, [r + _N])
            w2 = plsc.load_gather(tbl_v, [s + 2 * _N])
            w3 = plsc.load_gather(tbl_v, [s + 3 * _N])
            pairs = ((_lo(w0), _lo(w2)), (_hi(w0), _hi(w2)),
                     (_lo(w1), _lo(w3)), (_hi(w1), _hi(w3)))
            for j, (a, bb) in enumerate(pairs):
                out_v[pl.ds((j // 2) * (2 * _CH) + be + (j % 2) * 128, 16)] = (
                    _tanh16(a, bb))

        out_dmas[b] = tuple(
            pltpu.async_copy(out_v.at[pl.ds(j1 * 2 * _CH, 2 * _CH)],
                             out_hbm.at[pl.ds(j1 * 2 * _E + 2 * off, 2 * _CH)],
                             osems[b])
            for j1 in range(2))

    for hs in out_dmas:
        if hs is not None:
            for h in hs:
                h.wait()


def kernel(x, edge_index, W):
    w1t = W[:, :_D].T
    w2t = W[:, _D:].T
    wc = jnp.concatenate([w1t, w2t], axis=1)
    wct = wc[:, jnp.array([0, 2, 4, 6, 1, 3, 5, 7])].T
    ypk2 = pl.pallas_call(
        _mm_body,
        out_shape=jax.ShapeDtypeStruct((_PKW, _N), jnp.int32),
    )(wct, x)
    out_t = _edge_maps(ypk2.reshape(-1), edge_index.reshape(-1))
    return (out_t.reshape(2, _E // 128, 2, 128)
            .transpose(1, 3, 0, 2).reshape(_E, 2, 2))


# speedup vs baseline: 1.1935x; 1.0084x over previous
import functools

import jax
import jax.numpy as jnp
from jax import lax
from jax.experimental import pallas as pl
from jax.experimental.pallas import tpu as pltpu
from jax.experimental.pallas import tpu_sc as plsc

_N = 10000
_D = 128
_E = 320000
_F = 4
_TBLW = 2 * _F
_PKW = _TBLW // 2

_NC = 2
_NS = 16
_NW = _NC * _NS
_WSTRIDE = 9984
_CH = 5248
_NCHUNK = 2

_HIMASK = -65536


def _mm_body(w_ref, x_ref, o_ref):
    yt = jax.lax.dot_general(
        w_ref[...], x_ref[...],
        dimension_numbers=(((1,), (1,)), ((), ())),
        preferred_element_type=jnp.float32)
    b = jax.lax.bitcast_convert_type(yt, jnp.uint32) + jnp.uint32(0x8000)
    lo = b[:_PKW, :] >> 16
    hi = b[_PKW:, :] & jnp.uint32(0xFFFF0000)
    o_ref[...] = jax.lax.bitcast_convert_type(lo | hi, jnp.int32)


def _tanh16(a, b):
    e = jnp.exp((a + b) * 2.0)
    w = 1.0 / (e + 1.0)
    return 1.0 - (w + w)


def _lo(w):
    return plsc.bitcast(w << 16, jnp.float32)


def _hi(w):
    return plsc.bitcast(w & _HIMASK, jnp.float32)


_mesh = plsc.VectorSubcoreMesh(core_axis_name="c", subcore_axis_name="s")


@functools.partial(
    pl.kernel,
    mesh=_mesh,
    out_type=jax.ShapeDtypeStruct((_F * _E,), jnp.float32),
    compiler_params=pltpu.CompilerParams(needs_layout_passes=False),
    scratch_types=[
        pltpu.VMEM((_N * _PKW,), jnp.int32),
        pltpu.VMEM((_CH,), jnp.int32),
        pltpu.VMEM((_CH,), jnp.int32),
        pltpu.VMEM((_CH,), jnp.int32),
        pltpu.VMEM((_CH,), jnp.int32),
        pltpu.VMEM((_F * _CH,), jnp.float32),
        pltpu.VMEM((_F * _CH,), jnp.float32),
        pltpu.SemaphoreType.DMA,
        pltpu.SemaphoreType.DMA,
        pltpu.SemaphoreType.DMA,
        pltpu.SemaphoreType.DMA,
        pltpu.SemaphoreType.DMA,
    ],
)
def _edge_maps(ypk_hbm, ei_hbm, out_hbm, tbl_v, r0, c0, r1, c1, o0, o1,
               sem_t, sem_i0, sem_i1, sem_o0, sem_o1):
    wid = lax.axis_index("s") * _NC + lax.axis_index("c")
    base = wid * _WSTRIDE
    rbufs = (r0, r1)
    cbufs = (c0, c1)
    obufs = (o0, o1)
    isems = (sem_i0, sem_i1)
    osems = (sem_o0, sem_o1)

    tbl_dma = pltpu.async_copy(ypk_hbm, tbl_v, sem_t)

    def _start_idx(c):
        b = c % 2
        off = base + c * _CH
        hr = pltpu.async_copy(ei_hbm.at[pl.ds(off, _CH)], rbufs[b], isems[b])
        hc = pltpu.async_copy(ei_hbm.at[pl.ds(_E + off, _CH)], cbufs[b],
                              isems[b])
        return hr, hc

    idx_dmas = [None, None]
    out_dmas = [None, None]
    idx_dmas[0] = _start_idx(0)
    tbl_dma.wait()

    for c in range(_NCHUNK):
        b = c % 2
        off = base + c * _CH
        if c + 1 < _NCHUNK:
            idx_dmas[1 - b] = _start_idx(c + 1)
        for h in idx_dmas[b]:
            h.wait()
        if out_dmas[b] is not None:
            for h in out_dmas[b]:
                h.wait()
        rows_v = rbufs[b]
        cols_v = cbufs[b]
        out_v = obufs[b]

        @plsc.parallel_loop(0, _CH, 16, unroll=2)
        def _step(i):
            be = (i // 128) * 256 + (i % 128)
            r = rows_v[pl.ds(i, 16)]
            s = cols_v[pl.ds(i, 16)]
            w0 = plsc.load_gather(tbl_v, [r])
            w1 = plsc.load_gather(tbl_v, [r + _N])
            w2 = plsc.load_gather(tbl_v, [s + 2 * _N])
            w3 = plsc.load_gather(tbl_v, [s + 3 * _N])
            pairs = ((_lo(w0), _lo(w2)), (_hi(w0), _hi(w2)),
                     (_lo(w1), _lo(w3)), (_hi(w1), _hi(w3)))
            for j, (a, bb) in enumerate(pairs):
                out_v[pl.ds((j // 2) * (2 * _CH) + be + (j % 2) * 128, 16)] = (
                    _tanh16(a, bb))

        out_dmas[b] = tuple(
            pltpu.async_copy(out_v.at[pl.ds(j1 * 2 * _CH, 2 * _CH)],
                             out_hbm.at[pl.ds(j1 * 2 * _E + 2 * off, 2 * _CH)],
                             osems[b])
            for j1 in range(2))

    for hs in out_dmas:
        if hs is not None:
            for h in hs:
                h.wait()


def kernel(x, edge_index, W):
    w1t = W[:, :_D].T
    w2t = W[:, _D:].T
    wc = jnp.concatenate([w1t, w2t], axis=1)
    wct = wc[:, jnp.array([0, 2, 4, 6, 1, 3, 5, 7])].T
    ypk2 = pl.pallas_call(
        _mm_body,
        out_shape=jax.ShapeDtypeStruct((_PKW, _N), jnp.int32),
    )(wct, x)
    out_t = _edge_maps(ypk2.reshape(-1), edge_index.reshape(-1))
    return (out_t.reshape(2, _E // 128, 2, 128)
            .transpose(1, 3, 0, 2).reshape(_E, 2, 2))
